# baseline (device time: 50466 ns/iter reference)
import os

import jax
import jax.numpy as jnp
from jax import lax
from jax.experimental import pallas as pl
from jax.experimental.pallas import tpu as pltpu

N_DEV = 4
SQ = 512
QPD = SQ // N_DEV
D = 1024
SKV = 2048
HQ = 8
HKV = 2
HPG = HQ // HKV
DH = 128
SCALE = 0.08838834764831843
_NOCOMM = os.environ.get("NOCOMM") == "1"


def _body(x_ref, wq_ref, wo_ref, k_ref, v_ref, out_ref,
          q_buf, o_loc, ml_loc, rs_o, rs_ml,
          rs_o_send, rs_o_recv, rs_ml_send, rs_ml_recv,
          ag_send, ag_recv):
    my = lax.axis_index("i")

    xb = x_ref[...].astype(jnp.bfloat16)
    q_buf[...] = (jnp.dot(xb, wq_ref[...].astype(jnp.bfloat16),
                          preferred_element_type=jnp.float32)
                  * SCALE).astype(jnp.bfloat16)
    kb = [k_ref[:, g * DH:(g + 1) * DH].astype(jnp.bfloat16)
          for g in range(HKV)]
    vb = [v_ref[:, g * DH:(g + 1) * DH].astype(jnp.bfloat16)
          for g in range(HKV)]

    def attn_block(rows, h):
        qb = q_buf[rows, h * DH:(h + 1) * DH]
        s = lax.dot_general(
            qb, kb[h // HPG], (((1,), (1,)), ((), ())),
            preferred_element_type=jnp.float32)
        m = jnp.max(s, axis=1, keepdims=True)
        p = jnp.exp(s - m)
        l = jnp.sum(p, axis=1, keepdims=True)
        o = jnp.dot(p.astype(jnp.bfloat16), vb[h // HPG],
                    preferred_element_type=jnp.float32)
        return o, m, l

    sends = []
    for st in (1, 2, 3):
        dst = lax.rem(my + st, N_DEV)
        rows = pl.ds(dst * QPD, QPD)
        for h in range(HQ):
            o, m, l = attn_block(rows, h)
            o_loc[rows, h * DH:(h + 1) * DH] = o.astype(jnp.bfloat16)
            ml_loc[rows, h:h + 1] = m
            ml_loc[rows, HQ + h:HQ + h + 1] = l
        slot = N_DEV - st
        for src_buf, dst_buf, ssem, rsem in () if _NOCOMM else (
                (o_loc, rs_o, rs_o_send, rs_o_recv),
                (ml_loc, rs_ml, rs_ml_send, rs_ml_recv)):
            r = pltpu.make_async_remote_copy(
                src_ref=src_buf.at[rows, :],
                dst_ref=dst_buf.at[slot],
                send_sem=ssem.at[st],
                recv_sem=rsem.at[slot],
                device_id=(dst,),
                device_id_type=pl.DeviceIdType.MESH,
            )
            r.start()
            sends.append(r)

    mine = pl.ds(my * QPD, QPD)
    O = [None] * HQ
    M = [None] * HQ
    L = [None] * HQ
    for h in range(HQ):
        O[h], M[h], L[h] = attn_block(mine, h)

    for j in () if _NOCOMM else (3, 2, 1):
        for src_buf, dst_buf, ssem, rsem in (
                (o_loc, rs_o, rs_o_send, rs_o_recv),
                (ml_loc, rs_ml, rs_ml_send, rs_ml_recv)):
            r = pltpu.make_async_remote_copy(
                src_ref=src_buf.at[pl.ds(0, QPD), :],
                dst_ref=dst_buf.at[j],
                send_sem=ssem.at[0],
                recv_sem=rsem.at[j],
                device_id=(my,),
                device_id_type=pl.DeviceIdType.MESH,
            )
            r.wait_recv()
        for h in range(HQ):
            m_j = rs_ml[j, :, h:h + 1]
            l_j = rs_ml[j, :, HQ + h:HQ + h + 1]
            o_j = rs_o[j, :, h * DH:(h + 1) * DH].astype(jnp.float32)
            m_new = jnp.maximum(M[h], m_j)
            a = jnp.exp(M[h] - m_new)
            b = jnp.exp(m_j - m_new)
            O[h] = O[h] * a + o_j * b
            L[h] = L[h] * a + l_j * b
            M[h] = m_new

    on_all = jnp.concatenate(
        [(O[h] / L[h]).astype(jnp.bfloat16) for h in range(HQ)],
        axis=1)
    acc = jnp.dot(on_all, wo_ref[...].astype(jnp.bfloat16),
                  preferred_element_type=jnp.float32)
    out_ref[mine, :] = acc.astype(jnp.bfloat16)

    for off in () if _NOCOMM else (1, 2, 3):
        dst = lax.rem(my + off, N_DEV)
        r = pltpu.make_async_remote_copy(
            src_ref=out_ref.at[mine, :],
            dst_ref=out_ref.at[mine, :],
            send_sem=ag_send.at[off],
            recv_sem=ag_recv.at[N_DEV - off],
            device_id=(dst,),
            device_id_type=pl.DeviceIdType.MESH,
        )
        r.start()
        sends.append(r)
    for j in () if _NOCOMM else (1, 2, 3):
        owner = lax.rem(my + (N_DEV - j), N_DEV)
        r = pltpu.make_async_remote_copy(
            src_ref=out_ref.at[mine, :],
            dst_ref=out_ref.at[pl.ds(owner * QPD, QPD), :],
            send_sem=ag_send.at[0],
            recv_sem=ag_recv.at[j],
            device_id=(my,),
            device_id_type=pl.DeviceIdType.MESH,
        )
        r.wait_recv()

    for r in sends:
        r.wait_send()


def kernel(x, Wq, Wo, K_ext, V_ext):
    x2 = x.reshape(SQ, D)
    k2 = K_ext.reshape(SKV, HKV * DH)
    v2 = V_ext.reshape(SKV, HKV * DH)
    out = pl.pallas_call(
        _body,
        out_shape=jax.ShapeDtypeStruct((SQ, D), jnp.bfloat16),
        in_specs=[pl.BlockSpec(memory_space=pltpu.VMEM)] * 5,
        out_specs=pl.BlockSpec(memory_space=pltpu.VMEM),
        scratch_shapes=[
            pltpu.VMEM((SQ, D), jnp.bfloat16),
            pltpu.VMEM((SQ, HQ * DH), jnp.bfloat16),
            pltpu.VMEM((SQ, 2 * HQ), jnp.float32),
            pltpu.VMEM((N_DEV, QPD, HQ * DH), jnp.bfloat16),
            pltpu.VMEM((N_DEV, QPD, 2 * HQ), jnp.float32),
            pltpu.SemaphoreType.DMA((N_DEV,)),
            pltpu.SemaphoreType.DMA((N_DEV,)),
            pltpu.SemaphoreType.DMA((N_DEV,)),
            pltpu.SemaphoreType.DMA((N_DEV,)),
            pltpu.SemaphoreType.DMA((N_DEV,)),
            pltpu.SemaphoreType.DMA((N_DEV,)),
        ],
        compiler_params=pltpu.CompilerParams(
            vmem_limit_bytes=100 * 1024 * 1024,
        ),
    )(x2, Wq, Wo, k2, v2)
    return out.reshape(1, SQ, D)
